# baseline (speedup 1.0000x reference)
import functools

import jax
import jax.numpy as jnp
from jax import lax
from jax.experimental import pallas as pl
from jax.experimental.pallas import tpu as pltpu
from jax.experimental.pallas import tpu_sc as plsc

_LANES = 16
_NUM_SUBCORES = 16


def _hist_body(idx_hbm, out_hbm, idx_v, hist_v, part_v, shared_v, red_v,
               sem0, sem1, sem2, num_experts, total_tokens, weight):
    row = idx_hbm.shape[1]
    q = row // 4
    sid = lax.axis_index("s")
    r0 = idx_hbm.at[2 * sid]
    c0 = pltpu.async_copy(r0.at[pl.ds(0, q)], idx_v.at[pl.ds(0, q)], sem0)
    c1 = pltpu.async_copy(r0.at[pl.ds(q, 3 * q)], idx_v.at[pl.ds(q, 3 * q)],
                          sem1)
    c2 = pltpu.async_copy(idx_hbm.at[2 * sid + 1], idx_v.at[pl.ds(row, row)],
                          sem2)

    zeros = jnp.zeros((_LANES,), jnp.float32)

    @plsc.parallel_loop(0, num_experts, unroll=4)
    def _zero(j):
        hist_v[pl.ds(j * _LANES, _LANES)] = zeros

    lane = lax.iota(jnp.int32, _LANES)
    ones = jnp.ones((_LANES,), jnp.float32)

    nvec = row // _LANES
    qvec = nvec // 4
    c0.wait()

    @plsc.parallel_loop(0, qvec, unroll=8)
    def _scatter0(i):
        v = idx_v[pl.ds(i * _LANES, _LANES)]
        plsc.addupdate_scatter(hist_v, [v * _LANES + lane], ones)

    c1.wait()

    @plsc.parallel_loop(qvec, nvec, unroll=8)
    def _scatter1(i):
        v = idx_v[pl.ds(i * _LANES, _LANES)]
        plsc.addupdate_scatter(hist_v, [v * _LANES + lane], ones)

    c2.wait()

    @plsc.parallel_loop(nvec, 2 * nvec, unroll=8)
    def _scatter2(i):
        v = idx_v[pl.ds(i * _LANES, _LANES)]
        plsc.addupdate_scatter(hist_v, [v * _LANES + lane], ones)

    @plsc.parallel_loop(0, num_experts // _LANES)
    def _reduce(g):
        eids = (g * _LANES + lane) * _LANES
        acc = jnp.zeros((_LANES,), jnp.float32)
        for l in range(_LANES):
            acc = acc + plsc.load_gather(hist_v, [eids + l])
        part_v[pl.ds(g * _LANES, _LANES)] = acc

    pltpu.sync_copy(part_v, shared_v.at[pl.ds(sid * num_experts, num_experts)])
    plsc.subcore_barrier()

    @pl.when(sid == 0)
    def _finalize():
        pltpu.sync_copy(shared_v, red_v)
        inv = 1.0 / total_tokens
        sq = jnp.zeros((_LANES,), jnp.float32)
        for g in range(num_experts // _LANES):
            tot = jnp.zeros((_LANES,), jnp.float32)
            for w in range(_NUM_SUBCORES):
                tot = tot + red_v[pl.ds(w * num_experts + g * _LANES, _LANES)]
            frac = tot * inv
            sq = sq + frac * frac
        loss = weight * (num_experts * jnp.sum(sq) - 1.0)
        part_v[pl.ds(0, _LANES)] = jnp.full((_LANES,), loss, jnp.float32)
        pltpu.sync_copy(part_v.at[pl.ds(0, _LANES)], out_hbm)


def kernel(gate_logits, top_k_indices):
    batch, seq, num_experts = gate_logits.shape
    total_tokens = batch * seq
    top_k = top_k_indices.shape[-1]
    n = batch * seq * top_k
    nrows = batch * top_k
    chunk = n // _NUM_SUBCORES
    idx = (
        jnp.transpose(top_k_indices.astype(jnp.int32), (0, 2, 1))
        .reshape(nrows, seq)
    )

    hist_fn = functools.partial(
        pl.kernel,
        mesh=plsc.VectorSubcoreMesh(
            core_axis_name="c", subcore_axis_name="s", num_cores=1
        ),
        out_type=jax.ShapeDtypeStruct((_LANES,), jnp.float32),
        scratch_types=[
            pltpu.VMEM((chunk,), jnp.int32),
            pltpu.VMEM((num_experts * _LANES,), jnp.float32),
            pltpu.VMEM((num_experts,), jnp.float32),
            pltpu.VMEM_SHARED((_NUM_SUBCORES * num_experts,), jnp.float32),
            pltpu.VMEM((_NUM_SUBCORES * num_experts,), jnp.float32),
            pltpu.SemaphoreType.DMA,
            pltpu.SemaphoreType.DMA,
            pltpu.SemaphoreType.DMA,
        ],
        compiler_params=pltpu.CompilerParams(needs_layout_passes=False),
    )(
        functools.partial(
            _hist_body,
            num_experts=num_experts,
            total_tokens=float(total_tokens),
            weight=0.01,
        )
    )
    out = hist_fn(idx)
    return out[0]


# speedup vs baseline: 1.0089x; 1.0089x over previous
import functools

import jax
import jax.numpy as jnp
from jax import lax
from jax.experimental import pallas as pl
from jax.experimental.pallas import tpu as pltpu
from jax.experimental.pallas import tpu_sc as plsc

_LANES = 16
_NUM_SUBCORES = 16


def _hist_body(idx_hbm, out_hbm, idx_v, hist_v, part_v, shared_v, red_v,
               sem0, sem1, num_experts, total_tokens, weight):
    row = idx_hbm.shape[1]
    sid = lax.axis_index("s")
    c0 = pltpu.async_copy(idx_hbm.at[2 * sid], idx_v.at[pl.ds(0, row)], sem0)
    c1 = pltpu.async_copy(idx_hbm.at[2 * sid + 1], idx_v.at[pl.ds(row, row)],
                          sem1)

    zeros = jnp.zeros((_LANES,), jnp.float32)

    @plsc.parallel_loop(0, num_experts, unroll=4)
    def _zero(j):
        hist_v[pl.ds(j * _LANES, _LANES)] = zeros

    lane = lax.iota(jnp.int32, _LANES)
    ones = jnp.ones((_LANES,), jnp.float32)

    nvec = row // _LANES
    c0.wait()

    @plsc.parallel_loop(0, nvec, unroll=8)
    def _scatter0(i):
        v = idx_v[pl.ds(i * _LANES, _LANES)]
        plsc.addupdate_scatter(hist_v, [v * _LANES + lane], ones)

    c1.wait()

    @plsc.parallel_loop(nvec, 2 * nvec, unroll=8)
    def _scatter1(i):
        v = idx_v[pl.ds(i * _LANES, _LANES)]
        plsc.addupdate_scatter(hist_v, [v * _LANES + lane], ones)

    @plsc.parallel_loop(0, num_experts // _LANES)
    def _reduce(g):
        eids = (g * _LANES + lane) * _LANES
        acc = jnp.zeros((_LANES,), jnp.float32)
        for l in range(_LANES):
            acc = acc + plsc.load_gather(hist_v, [eids + l])
        part_v[pl.ds(g * _LANES, _LANES)] = acc

    pltpu.sync_copy(part_v, shared_v.at[pl.ds(sid * num_experts, num_experts)])
    plsc.subcore_barrier()

    @pl.when(sid == 0)
    def _finalize():
        pltpu.sync_copy(shared_v, red_v)
        inv = 1.0 / total_tokens
        sq = jnp.zeros((_LANES,), jnp.float32)
        for g in range(num_experts // _LANES):
            tot = jnp.zeros((_LANES,), jnp.float32)
            for w in range(_NUM_SUBCORES):
                tot = tot + red_v[pl.ds(w * num_experts + g * _LANES, _LANES)]
            frac = tot * inv
            sq = sq + frac * frac
        loss = weight * (num_experts * jnp.sum(sq) - 1.0)
        part_v[pl.ds(0, _LANES)] = jnp.full((_LANES,), loss, jnp.float32)
        pltpu.sync_copy(part_v.at[pl.ds(0, _LANES)], out_hbm)


def kernel(gate_logits, top_k_indices):
    batch, seq, num_experts = gate_logits.shape
    total_tokens = batch * seq
    top_k = top_k_indices.shape[-1]
    n = batch * seq * top_k
    nrows = batch * top_k
    chunk = n // _NUM_SUBCORES
    idx = (
        jnp.transpose(top_k_indices.astype(jnp.int32), (0, 2, 1))
        .reshape(nrows, seq)
    )

    hist_fn = functools.partial(
        pl.kernel,
        mesh=plsc.VectorSubcoreMesh(
            core_axis_name="c", subcore_axis_name="s", num_cores=1
        ),
        out_type=jax.ShapeDtypeStruct((_LANES,), jnp.float32),
        scratch_types=[
            pltpu.VMEM((chunk,), jnp.int32),
            pltpu.VMEM((num_experts * _LANES,), jnp.float32),
            pltpu.VMEM((num_experts,), jnp.float32),
            pltpu.VMEM_SHARED((_NUM_SUBCORES * num_experts,), jnp.float32),
            pltpu.VMEM((_NUM_SUBCORES * num_experts,), jnp.float32),
            pltpu.SemaphoreType.DMA,
            pltpu.SemaphoreType.DMA,
        ],
        compiler_params=pltpu.CompilerParams(needs_layout_passes=False),
    )(
        functools.partial(
            _hist_body,
            num_experts=num_experts,
            total_tokens=float(total_tokens),
            weight=0.01,
        )
    )
    out = hist_fn(idx)
    return out[0]
